# Initial kernel scaffold; baseline (speedup 1.0000x reference)
import jax
import jax.numpy as jnp
from jax.experimental import pallas as pl


def kernel(x, edge_index, W0, b0, W1, b1, W2, b2, Wc, bc):
    raise NotImplementedError("write your pallas kernel here")


# speedup vs baseline: 9.6342x; 9.6342x over previous
import functools

import jax
import jax.numpy as jnp
from jax import lax
from jax.experimental import pallas as pl
from jax.experimental.pallas import tpu as pltpu
from jax.experimental.pallas import tpu_sc as plsc

N = 10000
NP = 10240
D = 128
E = 320000
CH = 128
NTILES = 32
NCH = 79
EPT = NCH * CH
EP = NTILES * EPT
DW = 16
RPS = NP // 16

_MESH = dict(core_axis_name="c", subcore_axis_name="s")


@functools.partial(
    pl.kernel,
    mesh=plsc.VectorSubcoreMesh(**_MESH),
    out_type=jax.ShapeDtypeStruct((2, NP, DW), jnp.float32),
    scratch_types=[
        pltpu.VMEM_SHARED((NP, DW), jnp.float32),
        pltpu.VMEM((CH,), jnp.int32),
        pltpu.VMEM((CH, DW), jnp.float32),
    ],
)
def _sc_degree(dst_hbm, ones_hbm, zeros_hbm, pdeg_hbm, acc_sh, idx_v, ones_v):
    c = lax.axis_index("c")
    s = lax.axis_index("s")
    w = s * 2 + c
    pltpu.sync_copy(zeros_hbm.at[pl.ds(s * RPS, RPS)],
                    acc_sh.at[pl.ds(s * RPS, RPS)])
    pltpu.sync_copy(ones_hbm, ones_v)
    plsc.subcore_barrier()

    def body(j, carry):
        base = w * EPT + j * CH
        pltpu.sync_copy(dst_hbm.at[pl.ds(base, CH)], idx_v)
        pltpu.sync_copy(ones_v, acc_sh.at[idx_v], add=True)
        return carry

    lax.fori_loop(0, NCH, body, 0)
    plsc.subcore_barrier()
    pltpu.sync_copy(acc_sh.at[pl.ds(s * RPS, RPS)],
                    pdeg_hbm.at[c, pl.ds(s * RPS, RPS)])


@functools.partial(
    pl.kernel,
    mesh=plsc.VectorSubcoreMesh(**_MESH),
    out_type=jax.ShapeDtypeStruct((2, NP, D), jnp.float32),
    scratch_types=[
        pltpu.VMEM_SHARED((NP, D), jnp.float32),
        pltpu.VMEM((CH,), jnp.int32),
        pltpu.VMEM((CH,), jnp.int32),
        pltpu.VMEM((CH, D), jnp.float32),
        pltpu.SemaphoreType.DMA,
    ],
)
def _sc_scatter(g_hbm, src_hbm, dst_hbm, p_hbm, acc_sh, sidx, didx, rows_v, sem):
    c = lax.axis_index("c")
    s = lax.axis_index("s")
    w = s * 2 + c
    pltpu.sync_copy(g_hbm.at[pl.ds(s * RPS, RPS)],
                    acc_sh.at[pl.ds(s * RPS, RPS)])
    plsc.subcore_barrier()

    def body(j, carry):
        base = w * EPT + j * CH
        pltpu.sync_copy(src_hbm.at[pl.ds(base, CH)], sidx)
        pltpu.sync_copy(dst_hbm.at[pl.ds(base, CH)], didx)
        pltpu.async_copy(g_hbm.at[sidx], rows_v, sem).wait()
        pltpu.sync_copy(rows_v, acc_sh.at[didx], add=True)
        return carry

    lax.fori_loop(0, NCH, body, 0)
    plsc.subcore_barrier()
    pltpu.sync_copy(acc_sh.at[pl.ds(s * RPS, RPS)],
                    p_hbm.at[c, pl.ds(s * RPS, RPS)])


def _dis_from_pdeg(pdeg_ref):
    degcol = pdeg_ref[0] + pdeg_ref[1]
    return lax.rsqrt(degcol[:, 0:1] + 1.0)


def _g0_body(x_ref, w_ref, pdeg_ref, g_ref):
    dis = _dis_from_pdeg(pdeg_ref)
    h = lax.dot_general(x_ref[...], w_ref[...], (((1,), (1,)), ((), ())),
                        preferred_element_type=jnp.float32)
    g_ref[...] = dis * h


def _tc_g0(xp, W0, pdeg):
    return pl.pallas_call(
        _g0_body,
        out_shape=jax.ShapeDtypeStruct((NP, D), jnp.float32),
    )(xp, W0, pdeg)


def _layer_body(relu, p_ref, g_ref, b_ref, pdeg_ref, wn_ref, out_ref):
    dis = _dis_from_pdeg(pdeg_ref)
    y = dis * (p_ref[0] + p_ref[1] - g_ref[...]) + b_ref[...]
    mask = lax.broadcasted_iota(jnp.int32, (NP, 1), 0) < N
    y = jnp.where(mask, y, 0.0)
    if relu:
        y = jnp.maximum(y, 0.0)
    m = jnp.sum(y, axis=0, keepdims=True) / N
    v = jnp.sum(y * y, axis=0, keepdims=True) / N - m * m
    yn = (y - m) * lax.rsqrt(v + 1e-5)
    h = lax.dot_general(yn, wn_ref[...], (((1,), (1,)), ((), ())),
                        preferred_element_type=jnp.float32)
    out_ref[...] = jnp.where(mask, dis * h, 0.0)


def _tc_layer(P, g, b2d, pdeg, Wn):
    return pl.pallas_call(
        functools.partial(_layer_body, True),
        out_shape=jax.ShapeDtypeStruct((NP, D), jnp.float32),
    )(P, g, b2d, pdeg, Wn)


def _final_body(p_ref, g_ref, b_ref, pdeg_ref, wc_ref, bc_ref, out_ref):
    dis = _dis_from_pdeg(pdeg_ref)
    y = dis * (p_ref[0] + p_ref[1] - g_ref[...]) + b_ref[...]
    mask = lax.broadcasted_iota(jnp.int32, (NP, 1), 0) < N
    y = jnp.where(mask, y, 0.0)
    m = jnp.sum(y, axis=0, keepdims=True) / N
    v = jnp.sum(y * y, axis=0, keepdims=True) / N - m * m
    yn = (y - m) * lax.rsqrt(v + 1e-5)
    yn = jnp.where(mask, yn, 0.0)
    pooled = jnp.sum(yn, axis=0, keepdims=True) / N
    logits = lax.dot_general(pooled, wc_ref[...], (((1,), (1,)), ((), ())),
                             preferred_element_type=jnp.float32)
    logits = logits + bc_ref[...]
    z = logits - jnp.max(logits, axis=1, keepdims=True)
    ez = jnp.exp(z)
    out_ref[...] = ez / jnp.sum(ez, axis=1, keepdims=True)


def _tc_final(P, g, b2d, pdeg, Wc, bc2d):
    return pl.pallas_call(
        _final_body,
        out_shape=jax.ShapeDtypeStruct((1, 10), jnp.float32),
    )(P, g, b2d, pdeg, Wc, bc2d)


def kernel(x, edge_index, W0, b0, W1, b1, W2, b2, Wc, bc):
    src = edge_index[0].astype(jnp.int32)
    dst = edge_index[1].astype(jnp.int32)
    pad = EP - E
    srcp = jnp.concatenate([src, jnp.zeros((pad,), jnp.int32)])
    dstp = jnp.concatenate([dst, jnp.full((pad,), N + 8, jnp.int32)])
    xp = jnp.pad(x, ((0, NP - N), (0, 0)))
    ones = jnp.ones((CH, DW), jnp.float32)
    zeros = jnp.zeros((NP, DW), jnp.float32)
    b0r, b1r, b2r, bcr = (b0.reshape(1, -1), b1.reshape(1, -1),
                          b2.reshape(1, -1), bc.reshape(1, -1))

    pdeg = _sc_degree(dstp, ones, zeros)
    g0 = _tc_g0(xp, W0, pdeg)
    P0 = _sc_scatter(g0, srcp, dstp)
    g1 = _tc_layer(P0, g0, b0r, pdeg, W1)
    P1 = _sc_scatter(g1, srcp, dstp)
    g2 = _tc_layer(P1, g1, b1r, pdeg, W2)
    P2 = _sc_scatter(g2, srcp, dstp)
    return _tc_final(P2, g2, b2r, pdeg, Wc, bcr)


# speedup vs baseline: 10.7929x; 1.1203x over previous
import functools

import jax
import jax.numpy as jnp
from jax import lax
from jax.experimental import pallas as pl
from jax.experimental.pallas import tpu as pltpu
from jax.experimental.pallas import tpu_sc as plsc

N = 10000
NP = 10240
D = 128
E = 320000
CH = 128
NTILES = 32
NCH = 79
EPT = NCH * CH
EP = NTILES * EPT
DW = 16
RPS = NP // 16

_MESH = dict(core_axis_name="c", subcore_axis_name="s")


@functools.partial(
    pl.kernel,
    mesh=plsc.VectorSubcoreMesh(**_MESH),
    out_type=jax.ShapeDtypeStruct((2, NP, DW), jnp.float32),
    scratch_types=[
        pltpu.VMEM_SHARED((NP, DW), jnp.float32),
        pltpu.VMEM((NCH, CH), jnp.int32),
        pltpu.VMEM((CH, DW), jnp.float32),
    ],
)
def _sc_degree(dst_hbm, ones_hbm, zeros_hbm, pdeg_hbm, acc_sh, didx_all, ones_v):
    c = lax.axis_index("c")
    s = lax.axis_index("s")
    w = s * 2 + c
    pltpu.sync_copy(zeros_hbm.at[pl.ds(s * RPS, RPS)],
                    acc_sh.at[pl.ds(s * RPS, RPS)])
    pltpu.sync_copy(dst_hbm.at[w], didx_all)
    pltpu.sync_copy(ones_hbm, ones_v)
    plsc.subcore_barrier()

    def body(j, carry):
        pltpu.sync_copy(ones_v, acc_sh.at[didx_all.at[j]], add=True)
        return carry

    lax.fori_loop(0, NCH, body, 0)
    plsc.subcore_barrier()
    pltpu.sync_copy(acc_sh.at[pl.ds(s * RPS, RPS)],
                    pdeg_hbm.at[c, pl.ds(s * RPS, RPS)])


@functools.partial(
    pl.kernel,
    mesh=plsc.VectorSubcoreMesh(**_MESH),
    out_type=jax.ShapeDtypeStruct((2, NP, D), jnp.float32),
    scratch_types=[
        pltpu.VMEM_SHARED((NP, D), jnp.float32),
        pltpu.VMEM((CH,), jnp.int32),
        pltpu.VMEM((CH,), jnp.int32),
        pltpu.VMEM((CH,), jnp.int32),
        pltpu.VMEM((CH,), jnp.int32),
        pltpu.VMEM((CH, D), jnp.float32),
        pltpu.VMEM((CH, D), jnp.float32),
        pltpu.SemaphoreType.DMA,
        pltpu.SemaphoreType.DMA,
        pltpu.SemaphoreType.DMA,
        pltpu.SemaphoreType.DMA,
        pltpu.SemaphoreType.DMA,
        pltpu.SemaphoreType.DMA,
    ],
)
def _sc_scatter(g_hbm, src_hbm, dst_hbm, p_hbm, acc_sh, s_a, s_b, d_a, d_b,
                rows_a, rows_b, gsem_a, gsem_b, ssem_a, ssem_b, dsem_a, dsem_b):
    c = lax.axis_index("c")
    s = lax.axis_index("s")
    w = s * 2 + c
    pltpu.sync_copy(g_hbm.at[pl.ds(s * RPS, RPS)],
                    acc_sh.at[pl.ds(s * RPS, RPS)])

    def _sidx(j, buf, sem):
        pltpu.async_copy(src_hbm.at[w, j], buf, sem)

    def _didx(j, buf, sem):
        pltpu.async_copy(dst_hbm.at[w, j], buf, sem)

    def _iwait(hbm, buf, sem):
        pltpu.make_async_copy(hbm.at[0, 0], buf, sem).wait()

    def _gather(sbuf, rows, sem):
        pltpu.async_copy(g_hbm.at[sbuf], rows, sem)

    def _gwait(rows, sem):
        pltpu.make_async_copy(g_hbm.at[pl.ds(0, CH)], rows, sem).wait()

    def _scat(rows, dbuf):
        pltpu.sync_copy(rows, acc_sh.at[dbuf], add=True)

    _sidx(0, s_a, ssem_a)
    _didx(0, d_a, dsem_a)
    plsc.subcore_barrier()
    _iwait(src_hbm, s_a, ssem_a)
    _gather(s_a, rows_a, gsem_a)
    _sidx(1, s_b, ssem_b)
    _didx(1, d_b, dsem_b)

    def body(i, carry):
        ja = 2 * i + 1
        jb = 2 * i + 2
        jn = 2 * i + 3
        _iwait(src_hbm, s_b, ssem_b)
        _gwait(rows_a, gsem_a)
        _gather(s_b, rows_b, gsem_b)
        _sidx(jb, s_a, ssem_a)
        _iwait(dst_hbm, d_a, dsem_a)
        _scat(rows_a, d_a)
        _didx(jb, d_a, dsem_a)
        _iwait(src_hbm, s_a, ssem_a)
        _gwait(rows_b, gsem_b)
        _gather(s_a, rows_a, gsem_a)

        @pl.when(jn < NCH)
        def _():
            _sidx(jn, s_b, ssem_b)

        _iwait(dst_hbm, d_b, dsem_b)
        _scat(rows_b, d_b)

        @pl.when(jn < NCH)
        def _():
            _didx(jn, d_b, dsem_b)

        return carry

    lax.fori_loop(0, (NCH - 1) // 2, body, 0)
    _gwait(rows_a, gsem_a)
    _iwait(dst_hbm, d_a, dsem_a)
    _scat(rows_a, d_a)
    plsc.subcore_barrier()
    pltpu.sync_copy(acc_sh.at[pl.ds(s * RPS, RPS)],
                    p_hbm.at[c, pl.ds(s * RPS, RPS)])


def _dis_from_pdeg(pdeg_ref):
    degcol = pdeg_ref[0] + pdeg_ref[1]
    return lax.rsqrt(degcol[:, 0:1] + 1.0)


def _g0_body(x_ref, w_ref, pdeg_ref, g_ref):
    dis = _dis_from_pdeg(pdeg_ref)
    h = lax.dot_general(x_ref[...], w_ref[...], (((1,), (1,)), ((), ())),
                        preferred_element_type=jnp.float32)
    g_ref[...] = dis * h


def _tc_g0(xp, W0, pdeg):
    return pl.pallas_call(
        _g0_body,
        out_shape=jax.ShapeDtypeStruct((NP, D), jnp.float32),
    )(xp, W0, pdeg)


def _layer_body(relu, p_ref, g_ref, b_ref, pdeg_ref, wn_ref, out_ref):
    dis = _dis_from_pdeg(pdeg_ref)
    y = dis * (p_ref[0] + p_ref[1] - g_ref[...]) + b_ref[...]
    mask = lax.broadcasted_iota(jnp.int32, (NP, 1), 0) < N
    y = jnp.where(mask, y, 0.0)
    if relu:
        y = jnp.maximum(y, 0.0)
    m = jnp.sum(y, axis=0, keepdims=True) / N
    v = jnp.sum(y * y, axis=0, keepdims=True) / N - m * m
    yn = (y - m) * lax.rsqrt(v + 1e-5)
    h = lax.dot_general(yn, wn_ref[...], (((1,), (1,)), ((), ())),
                        preferred_element_type=jnp.float32)
    out_ref[...] = jnp.where(mask, dis * h, 0.0)


def _tc_layer(P, g, b2d, pdeg, Wn):
    return pl.pallas_call(
        functools.partial(_layer_body, True),
        out_shape=jax.ShapeDtypeStruct((NP, D), jnp.float32),
    )(P, g, b2d, pdeg, Wn)


def _final_body(p_ref, g_ref, b_ref, pdeg_ref, wc_ref, bc_ref, out_ref):
    dis = _dis_from_pdeg(pdeg_ref)
    y = dis * (p_ref[0] + p_ref[1] - g_ref[...]) + b_ref[...]
    mask = lax.broadcasted_iota(jnp.int32, (NP, 1), 0) < N
    y = jnp.where(mask, y, 0.0)
    m = jnp.sum(y, axis=0, keepdims=True) / N
    v = jnp.sum(y * y, axis=0, keepdims=True) / N - m * m
    yn = (y - m) * lax.rsqrt(v + 1e-5)
    yn = jnp.where(mask, yn, 0.0)
    pooled = jnp.sum(yn, axis=0, keepdims=True) / N
    logits = lax.dot_general(pooled, wc_ref[...], (((1,), (1,)), ((), ())),
                             preferred_element_type=jnp.float32)
    logits = logits + bc_ref[...]
    z = logits - jnp.max(logits, axis=1, keepdims=True)
    ez = jnp.exp(z)
    out_ref[...] = ez / jnp.sum(ez, axis=1, keepdims=True)


def _tc_final(P, g, b2d, pdeg, Wc, bc2d):
    return pl.pallas_call(
        _final_body,
        out_shape=jax.ShapeDtypeStruct((1, 10), jnp.float32),
    )(P, g, b2d, pdeg, Wc, bc2d)


def kernel(x, edge_index, W0, b0, W1, b1, W2, b2, Wc, bc):
    src = edge_index[0].astype(jnp.int32)
    dst = edge_index[1].astype(jnp.int32)
    pad = EP - E
    srcp = jnp.concatenate([src, jnp.zeros((pad,), jnp.int32)])
    dstp = jnp.concatenate([dst, jnp.full((pad,), N + 8, jnp.int32)])
    srcp = srcp.reshape(NTILES, NCH, CH)
    dstp = dstp.reshape(NTILES, NCH, CH)
    xp = jnp.pad(x, ((0, NP - N), (0, 0)))
    ones = jnp.ones((CH, DW), jnp.float32)
    zeros = jnp.zeros((NP, DW), jnp.float32)
    b0r, b1r, b2r, bcr = (b0.reshape(1, -1), b1.reshape(1, -1),
                          b2.reshape(1, -1), bc.reshape(1, -1))

    pdeg = _sc_degree(dstp, ones, zeros)
    g0 = _tc_g0(xp, W0, pdeg)
    P0 = _sc_scatter(g0, srcp, dstp)
    g1 = _tc_layer(P0, g0, b0r, pdeg, W1)
    P1 = _sc_scatter(g1, srcp, dstp)
    g2 = _tc_layer(P1, g1, b1r, pdeg, W2)
    P2 = _sc_scatter(g2, srcp, dstp)
    return _tc_final(P2, g2, b2r, pdeg, Wc, bcr)


# speedup vs baseline: 11.6646x; 1.0808x over previous
import functools

import jax
import jax.numpy as jnp
from jax import lax
from jax.experimental import pallas as pl
from jax.experimental.pallas import tpu as pltpu
from jax.experimental.pallas import tpu_sc as plsc

N = 10000
NP = 10240
D = 128
E = 320000
CH = 128
NTILES = 32
NCH = 79
EPT = NCH * CH
EP = NTILES * EPT
DW = 16
RPS = NP // 16
NCHG = NTILES * NCH
CN0 = 45
CN1 = 113

_MESH = dict(core_axis_name="c", subcore_axis_name="s")


@functools.partial(
    pl.kernel,
    mesh=plsc.VectorSubcoreMesh(**_MESH),
    out_type=jax.ShapeDtypeStruct((2, NP, DW), jnp.float32),
    scratch_types=[
        pltpu.VMEM_SHARED((NP, DW), jnp.float32),
        pltpu.VMEM((CH,), jnp.int32),
        pltpu.VMEM((CH, DW), jnp.float32),
    ],
)
def _sc_degree(dst_hbm, ones_hbm, zeros_hbm, pdeg_hbm, acc_sh, didx, ones_v):
    c = lax.axis_index("c")
    s = lax.axis_index("s")
    w = s * 2 + c
    pltpu.sync_copy(zeros_hbm.at[pl.ds(s * RPS, RPS)],
                    acc_sh.at[pl.ds(s * RPS, RPS)])
    pltpu.sync_copy(ones_hbm, ones_v)
    plsc.subcore_barrier()

    def body(j, carry):
        pltpu.sync_copy(dst_hbm.at[pl.ds(w * EPT + j * CH, CH)], didx)
        pltpu.sync_copy(ones_v, acc_sh.at[didx], add=True)
        return carry

    lax.fori_loop(0, NCH, body, 0)
    plsc.subcore_barrier()
    pltpu.sync_copy(acc_sh.at[pl.ds(s * RPS, RPS)],
                    pdeg_hbm.at[c, pl.ds(s * RPS, RPS)])


@functools.partial(
    pl.kernel,
    mesh=plsc.VectorSubcoreMesh(**_MESH),
    out_type=jax.ShapeDtypeStruct((2, NP, D), jnp.float32),
    scratch_types=[
        pltpu.VMEM_SHARED((NP, D), jnp.float32),
        pltpu.VMEM((CH,), jnp.int32),
        pltpu.VMEM((CH,), jnp.int32),
        pltpu.VMEM((CH,), jnp.int32),
        pltpu.VMEM((CH,), jnp.int32),
        pltpu.VMEM((CH, D), jnp.float32),
        pltpu.VMEM((CH, D), jnp.float32),
        pltpu.SemaphoreType.DMA,
        pltpu.SemaphoreType.DMA,
        pltpu.SemaphoreType.DMA,
        pltpu.SemaphoreType.DMA,
        pltpu.SemaphoreType.DMA,
        pltpu.SemaphoreType.DMA,
    ],
)
def _sc_scatter(g_hbm, src_hbm, dst_hbm, p_hbm, acc_sh, s_a, s_b, d_a, d_b,
                rows_a, rows_b, gsem_a, gsem_b, ssem_a, ssem_b, dsem_a, dsem_b):
    c = lax.axis_index("c")
    s = lax.axis_index("s")
    start = jnp.where(c == 0, s * CN0, 16 * CN0 + s * CN1)
    cnt = jnp.where(c == 0, CN0, CN1)
    pltpu.sync_copy(g_hbm.at[pl.ds(s * RPS, RPS)],
                    acc_sh.at[pl.ds(s * RPS, RPS)])

    def _sidx(j, buf, sem):
        pltpu.async_copy(src_hbm.at[pl.ds((start + j) * CH, CH)], buf, sem)

    def _didx(j, buf, sem):
        pltpu.async_copy(dst_hbm.at[pl.ds((start + j) * CH, CH)], buf, sem)

    def _iwait(hbm, buf, sem):
        pltpu.make_async_copy(hbm.at[pl.ds(0, CH)], buf, sem).wait()

    def _gather(sbuf, rows, sem):
        pltpu.async_copy(g_hbm.at[sbuf], rows, sem)

    def _gwait(rows, sem):
        pltpu.make_async_copy(g_hbm.at[pl.ds(0, CH)], rows, sem).wait()

    def _scat(rows, dbuf):
        pltpu.sync_copy(rows, acc_sh.at[dbuf], add=True)

    _sidx(0, s_a, ssem_a)
    _didx(0, d_a, dsem_a)
    plsc.subcore_barrier()
    _iwait(src_hbm, s_a, ssem_a)
    _gather(s_a, rows_a, gsem_a)
    _sidx(1, s_b, ssem_b)
    _didx(1, d_b, dsem_b)

    def body(i, carry):
        ja = 2 * i + 1
        jb = 2 * i + 2
        jn = 2 * i + 3
        _iwait(src_hbm, s_b, ssem_b)
        _gwait(rows_a, gsem_a)
        _gather(s_b, rows_b, gsem_b)
        _sidx(jb, s_a, ssem_a)
        _iwait(dst_hbm, d_a, dsem_a)
        _scat(rows_a, d_a)
        _didx(jb, d_a, dsem_a)
        _iwait(src_hbm, s_a, ssem_a)
        _gwait(rows_b, gsem_b)
        _gather(s_a, rows_a, gsem_a)

        @pl.when(jn < cnt)
        def _():
            _sidx(jn, s_b, ssem_b)

        _iwait(dst_hbm, d_b, dsem_b)
        _scat(rows_b, d_b)

        @pl.when(jn < cnt)
        def _():
            _didx(jn, d_b, dsem_b)

        return carry

    lax.fori_loop(0, (cnt - 1) // 2, body, 0)
    _gwait(rows_a, gsem_a)
    _iwait(dst_hbm, d_a, dsem_a)
    _scat(rows_a, d_a)
    plsc.subcore_barrier()
    pltpu.sync_copy(acc_sh.at[pl.ds(s * RPS, RPS)],
                    p_hbm.at[c, pl.ds(s * RPS, RPS)])


def _dis_from_pdeg(pdeg_ref):
    degcol = pdeg_ref[0] + pdeg_ref[1]
    return lax.rsqrt(degcol[:, 0:1] + 1.0)


def _g0_body(x_ref, w_ref, pdeg_ref, g_ref):
    dis = _dis_from_pdeg(pdeg_ref)
    h = lax.dot_general(x_ref[...], w_ref[...], (((1,), (1,)), ((), ())),
                        preferred_element_type=jnp.float32)
    g_ref[...] = dis * h


def _tc_g0(xp, W0, pdeg):
    return pl.pallas_call(
        _g0_body,
        out_shape=jax.ShapeDtypeStruct((NP, D), jnp.float32),
    )(xp, W0, pdeg)


def _layer_body(relu, p_ref, g_ref, b_ref, pdeg_ref, wn_ref, out_ref):
    dis = _dis_from_pdeg(pdeg_ref)
    y = dis * (p_ref[0] + p_ref[1] - g_ref[...]) + b_ref[...]
    mask = lax.broadcasted_iota(jnp.int32, (NP, 1), 0) < N
    y = jnp.where(mask, y, 0.0)
    if relu:
        y = jnp.maximum(y, 0.0)
    m = jnp.sum(y, axis=0, keepdims=True) / N
    v = jnp.sum(y * y, axis=0, keepdims=True) / N - m * m
    yn = (y - m) * lax.rsqrt(v + 1e-5)
    h = lax.dot_general(yn, wn_ref[...], (((1,), (1,)), ((), ())),
                        preferred_element_type=jnp.float32)
    out_ref[...] = jnp.where(mask, dis * h, 0.0)


def _tc_layer(P, g, b2d, pdeg, Wn):
    return pl.pallas_call(
        functools.partial(_layer_body, True),
        out_shape=jax.ShapeDtypeStruct((NP, D), jnp.float32),
    )(P, g, b2d, pdeg, Wn)


def _final_body(p_ref, g_ref, b_ref, pdeg_ref, wc_ref, bc_ref, out_ref):
    dis = _dis_from_pdeg(pdeg_ref)
    y = dis * (p_ref[0] + p_ref[1] - g_ref[...]) + b_ref[...]
    mask = lax.broadcasted_iota(jnp.int32, (NP, 1), 0) < N
    y = jnp.where(mask, y, 0.0)
    m = jnp.sum(y, axis=0, keepdims=True) / N
    v = jnp.sum(y * y, axis=0, keepdims=True) / N - m * m
    yn = (y - m) * lax.rsqrt(v + 1e-5)
    yn = jnp.where(mask, yn, 0.0)
    pooled = jnp.sum(yn, axis=0, keepdims=True) / N
    logits = lax.dot_general(pooled, wc_ref[...], (((1,), (1,)), ((), ())),
                             preferred_element_type=jnp.float32)
    logits = logits + bc_ref[...]
    z = logits - jnp.max(logits, axis=1, keepdims=True)
    ez = jnp.exp(z)
    out_ref[...] = ez / jnp.sum(ez, axis=1, keepdims=True)


def _tc_final(P, g, b2d, pdeg, Wc, bc2d):
    return pl.pallas_call(
        _final_body,
        out_shape=jax.ShapeDtypeStruct((1, 10), jnp.float32),
    )(P, g, b2d, pdeg, Wc, bc2d)


def kernel(x, edge_index, W0, b0, W1, b1, W2, b2, Wc, bc):
    src = edge_index[0].astype(jnp.int32)
    dst = edge_index[1].astype(jnp.int32)
    pad = EP - E
    srcp = jnp.concatenate([src, jnp.zeros((pad,), jnp.int32)])
    dstp = jnp.concatenate([dst, jnp.full((pad,), N + 8, jnp.int32)])
    xp = jnp.pad(x, ((0, NP - N), (0, 0)))
    ones = jnp.ones((CH, DW), jnp.float32)
    zeros = jnp.zeros((NP, DW), jnp.float32)
    b0r, b1r, b2r, bcr = (b0.reshape(1, -1), b1.reshape(1, -1),
                          b2.reshape(1, -1), bc.reshape(1, -1))

    pdeg = _sc_degree(dstp, ones, zeros)
    g0 = _tc_g0(xp, W0, pdeg)
    P0 = _sc_scatter(g0, srcp, dstp)
    g1 = _tc_layer(P0, g0, b0r, pdeg, W1)
    P1 = _sc_scatter(g1, srcp, dstp)
    g2 = _tc_layer(P1, g1, b1r, pdeg, W2)
    P2 = _sc_scatter(g2, srcp, dstp)
    return _tc_final(P2, g2, b2r, pdeg, Wc, bcr)


# speedup vs baseline: 13.9592x; 1.1967x over previous
import functools

import jax
import jax.numpy as jnp
from jax import lax
from jax.experimental import pallas as pl
from jax.experimental.pallas import tpu as pltpu
from jax.experimental.pallas import tpu_sc as plsc

N = 10000
NP = 10240
D = 128
E = 320000
CH = 128
NTILES = 32
NCH = 79
EPT = NCH * CH
EP = NTILES * EPT
DW = 16
RPS = NP // 16
NCHG = NTILES * NCH
CN0 = 137
CN1 = 21

_MESH = dict(core_axis_name="c", subcore_axis_name="s")


@functools.partial(
    pl.kernel,
    mesh=plsc.VectorSubcoreMesh(**_MESH),
    out_type=jax.ShapeDtypeStruct((2, NP, DW), jnp.float32),
    scratch_types=[
        pltpu.VMEM_SHARED((NP, DW), jnp.float32),
        pltpu.VMEM((CH,), jnp.int32),
        pltpu.VMEM((CH, DW), jnp.float32),
    ],
)
def _sc_degree(dst_hbm, ones_hbm, zeros_hbm, pdeg_hbm, acc_sh, didx, ones_v):
    c = lax.axis_index("c")
    s = lax.axis_index("s")
    w = s * 2 + c
    pltpu.sync_copy(zeros_hbm.at[pl.ds(s * RPS, RPS)],
                    acc_sh.at[pl.ds(s * RPS, RPS)])
    pltpu.sync_copy(ones_hbm, ones_v)
    plsc.subcore_barrier()

    def body(j, carry):
        pltpu.sync_copy(dst_hbm.at[pl.ds(w * EPT + j * CH, CH)], didx)
        pltpu.sync_copy(ones_v, acc_sh.at[didx], add=True)
        return carry

    lax.fori_loop(0, NCH, body, 0)
    plsc.subcore_barrier()
    pltpu.sync_copy(acc_sh.at[pl.ds(s * RPS, RPS)],
                    pdeg_hbm.at[c, pl.ds(s * RPS, RPS)])


@functools.partial(
    pl.kernel,
    mesh=plsc.VectorSubcoreMesh(**_MESH),
    out_type=jax.ShapeDtypeStruct((2, NP, D), jnp.float32),
    scratch_types=[
        pltpu.VMEM_SHARED((NP, D), jnp.float32),
        pltpu.VMEM((CH,), jnp.int32),
        pltpu.VMEM((CH,), jnp.int32),
        pltpu.VMEM((CH,), jnp.int32),
        pltpu.VMEM((CH,), jnp.int32),
        pltpu.VMEM((CH, D), jnp.float32),
        pltpu.VMEM((CH, D), jnp.float32),
        pltpu.SemaphoreType.DMA,
        pltpu.SemaphoreType.DMA,
        pltpu.SemaphoreType.DMA,
        pltpu.SemaphoreType.DMA,
        pltpu.SemaphoreType.DMA,
        pltpu.SemaphoreType.DMA,
    ],
)
def _sc_scatter(g_hbm, src_hbm, dst_hbm, p_hbm, acc_sh, s_a, s_b, d_a, d_b,
                rows_a, rows_b, gsem_a, gsem_b, ssem_a, ssem_b, dsem_a, dsem_b):
    c = lax.axis_index("c")
    s = lax.axis_index("s")
    start = jnp.where(c == 0, s * CN0, 16 * CN0 + s * CN1)
    cnt = jnp.where(c == 0, CN0, CN1)
    pltpu.sync_copy(g_hbm.at[pl.ds(s * RPS, RPS)],
                    acc_sh.at[pl.ds(s * RPS, RPS)])

    def _sidx(j, buf, sem):
        pltpu.async_copy(src_hbm.at[pl.ds((start + j) * CH, CH)], buf, sem)

    def _didx(j, buf, sem):
        pltpu.async_copy(dst_hbm.at[pl.ds((start + j) * CH, CH)], buf, sem)

    def _iwait(hbm, buf, sem):
        pltpu.make_async_copy(hbm.at[pl.ds(0, CH)], buf, sem).wait()

    def _gather(sbuf, rows, sem):
        pltpu.async_copy(g_hbm.at[sbuf], rows, sem)

    def _gwait(rows, sem):
        pltpu.make_async_copy(g_hbm.at[pl.ds(0, CH)], rows, sem).wait()

    def _scat(rows, dbuf):
        pltpu.sync_copy(rows, acc_sh.at[dbuf], add=True)

    _sidx(0, s_a, ssem_a)
    _didx(0, d_a, dsem_a)
    plsc.subcore_barrier()
    _iwait(src_hbm, s_a, ssem_a)
    _gather(s_a, rows_a, gsem_a)
    _sidx(1, s_b, ssem_b)
    _didx(1, d_b, dsem_b)

    def body(i, carry):
        ja = 2 * i + 1
        jb = 2 * i + 2
        jn = 2 * i + 3
        _iwait(src_hbm, s_b, ssem_b)
        _gwait(rows_a, gsem_a)
        _gather(s_b, rows_b, gsem_b)
        _sidx(jb, s_a, ssem_a)
        _iwait(dst_hbm, d_a, dsem_a)
        _scat(rows_a, d_a)
        _didx(jb, d_a, dsem_a)
        _iwait(src_hbm, s_a, ssem_a)
        _gwait(rows_b, gsem_b)
        _gather(s_a, rows_a, gsem_a)

        @pl.when(jn < cnt)
        def _():
            _sidx(jn, s_b, ssem_b)

        _iwait(dst_hbm, d_b, dsem_b)
        _scat(rows_b, d_b)

        @pl.when(jn < cnt)
        def _():
            _didx(jn, d_b, dsem_b)

        return carry

    lax.fori_loop(0, (cnt - 1) // 2, body, 0)
    _gwait(rows_a, gsem_a)
    _iwait(dst_hbm, d_a, dsem_a)
    _scat(rows_a, d_a)
    plsc.subcore_barrier()
    pltpu.sync_copy(acc_sh.at[pl.ds(s * RPS, RPS)],
                    p_hbm.at[c, pl.ds(s * RPS, RPS)])


def _dis_from_pdeg(pdeg_ref):
    degcol = pdeg_ref[0] + pdeg_ref[1]
    return lax.rsqrt(degcol[:, 0:1] + 1.0)


def _g0_body(x_ref, w_ref, pdeg_ref, g_ref):
    dis = _dis_from_pdeg(pdeg_ref)
    h = lax.dot_general(x_ref[...], w_ref[...], (((1,), (1,)), ((), ())),
                        preferred_element_type=jnp.float32)
    g_ref[...] = dis * h


def _tc_g0(xp, W0, pdeg):
    return pl.pallas_call(
        _g0_body,
        out_shape=jax.ShapeDtypeStruct((NP, D), jnp.float32),
    )(xp, W0, pdeg)


def _layer_body(relu, p_ref, g_ref, b_ref, pdeg_ref, wn_ref, out_ref):
    dis = _dis_from_pdeg(pdeg_ref)
    y = dis * (p_ref[0] + p_ref[1] - g_ref[...]) + b_ref[...]
    mask = lax.broadcasted_iota(jnp.int32, (NP, 1), 0) < N
    y = jnp.where(mask, y, 0.0)
    if relu:
        y = jnp.maximum(y, 0.0)
    m = jnp.sum(y, axis=0, keepdims=True) / N
    v = jnp.sum(y * y, axis=0, keepdims=True) / N - m * m
    yn = (y - m) * lax.rsqrt(v + 1e-5)
    h = lax.dot_general(yn, wn_ref[...], (((1,), (1,)), ((), ())),
                        preferred_element_type=jnp.float32)
    out_ref[...] = jnp.where(mask, dis * h, 0.0)


def _tc_layer(P, g, b2d, pdeg, Wn):
    return pl.pallas_call(
        functools.partial(_layer_body, True),
        out_shape=jax.ShapeDtypeStruct((NP, D), jnp.float32),
    )(P, g, b2d, pdeg, Wn)


def _final_body(p_ref, g_ref, b_ref, pdeg_ref, wc_ref, bc_ref, out_ref):
    dis = _dis_from_pdeg(pdeg_ref)
    y = dis * (p_ref[0] + p_ref[1] - g_ref[...]) + b_ref[...]
    mask = lax.broadcasted_iota(jnp.int32, (NP, 1), 0) < N
    y = jnp.where(mask, y, 0.0)
    m = jnp.sum(y, axis=0, keepdims=True) / N
    v = jnp.sum(y * y, axis=0, keepdims=True) / N - m * m
    yn = (y - m) * lax.rsqrt(v + 1e-5)
    yn = jnp.where(mask, yn, 0.0)
    pooled = jnp.sum(yn, axis=0, keepdims=True) / N
    logits = lax.dot_general(pooled, wc_ref[...], (((1,), (1,)), ((), ())),
                             preferred_element_type=jnp.float32)
    logits = logits + bc_ref[...]
    z = logits - jnp.max(logits, axis=1, keepdims=True)
    ez = jnp.exp(z)
    out_ref[...] = ez / jnp.sum(ez, axis=1, keepdims=True)


def _tc_final(P, g, b2d, pdeg, Wc, bc2d):
    return pl.pallas_call(
        _final_body,
        out_shape=jax.ShapeDtypeStruct((1, 10), jnp.float32),
    )(P, g, b2d, pdeg, Wc, bc2d)


def kernel(x, edge_index, W0, b0, W1, b1, W2, b2, Wc, bc):
    src = edge_index[0].astype(jnp.int32)
    dst = edge_index[1].astype(jnp.int32)
    pad = EP - E
    srcp = jnp.concatenate([src, jnp.zeros((pad,), jnp.int32)])
    dstp = jnp.concatenate([dst, jnp.full((pad,), N + 8, jnp.int32)])
    xp = jnp.pad(x, ((0, NP - N), (0, 0)))
    ones = jnp.ones((CH, DW), jnp.float32)
    zeros = jnp.zeros((NP, DW), jnp.float32)
    b0r, b1r, b2r, bcr = (b0.reshape(1, -1), b1.reshape(1, -1),
                          b2.reshape(1, -1), bc.reshape(1, -1))

    pdeg = _sc_degree(dstp, ones, zeros)
    g0 = _tc_g0(xp, W0, pdeg)
    P0 = _sc_scatter(g0, srcp, dstp)
    g1 = _tc_layer(P0, g0, b0r, pdeg, W1)
    P1 = _sc_scatter(g1, srcp, dstp)
    g2 = _tc_layer(P1, g1, b1r, pdeg, W2)
    P2 = _sc_scatter(g2, srcp, dstp)
    return _tc_final(P2, g2, b2r, pdeg, Wc, bcr)
